# baseline (speedup 1.0000x reference)
import functools

import jax
import jax.numpy as jnp
from jax import lax
from jax.experimental import pallas as pl
from jax.experimental.pallas import tpu as pltpu
from jax.experimental.pallas import tpu_sc as plsc

B, C, H, W = 32, 32, 224, 224

_mesh = plsc.VectorSubcoreMesh(core_axis_name="c", subcore_axis_name="s")


@functools.partial(
    pl.kernel,
    out_type=jax.ShapeDtypeStruct((B * C, H, W), jnp.float32),
    mesh=_mesh,
    scratch_types=[
        pltpu.VMEM((B + 16,), jnp.int32),
        pltpu.VMEM((2, H, W), jnp.float32),
        pltpu.SemaphoreType.DMA,
        pltpu.SemaphoreType.DMA,
    ],
)
def _sc_gather_bcast(tab_hbm, idx_hbm, out_hbm, idx_v, plane_v, gsem, wsem):
    wid = lax.axis_index("s") * 2 + lax.axis_index("c")
    pltpu.sync_copy(idx_hbm, idx_v.at[pl.ds(0, B)])
    src = wid * C + idx_v[pl.ds(wid, 16)][0]
    g0 = pltpu.async_copy(tab_hbm.at[pl.ds(src, 1)], plane_v.at[pl.ds(0, 1)], gsem)
    g1 = pltpu.async_copy(tab_hbm.at[pl.ds(src, 1)], plane_v.at[pl.ds(1, 1)], gsem)
    g0.wait()
    g1.wait()
    copies = [
        pltpu.async_copy(plane_v, out_hbm.at[pl.ds(wid * C + 2 * c, 2)], wsem)
        for c in range(C // 2)
    ]
    for cp in copies:
        cp.wait()


def kernel(logits, indices):
    tab = logits.reshape(B * C, H, W)
    idx = indices.astype(jnp.int32)
    out = _sc_gather_bcast(tab, idx)
    return out.reshape(B, C, H, W)


# speedup vs baseline: 1.0130x; 1.0130x over previous
import functools

import jax
import jax.numpy as jnp
from jax import lax
from jax.experimental import pallas as pl
from jax.experimental.pallas import tpu as pltpu
from jax.experimental.pallas import tpu_sc as plsc

B, C, H, W = 32, 32, 224, 224

_mesh = plsc.VectorSubcoreMesh(core_axis_name="c", subcore_axis_name="s")


@functools.partial(
    pl.kernel,
    out_type=jax.ShapeDtypeStruct((B * C, H, W), jnp.float32),
    mesh=_mesh,
    scratch_types=[
        pltpu.VMEM((B + 16,), jnp.int32),
        pltpu.VMEM((1, H, W), jnp.float32),
        pltpu.SemaphoreType.DMA,
        pltpu.SemaphoreType.DMA,
        pltpu.SemaphoreType.DMA,
    ],
)
def _sc_gather_bcast(tab_hbm, idx_hbm, out_hbm, idx_v, plane_v, gsem, g2sem, wsem):
    wid = lax.axis_index("s") * 2 + lax.axis_index("c")
    pltpu.sync_copy(idx_hbm, idx_v.at[pl.ds(0, B)])
    src = wid * C + idx_v[pl.ds(wid, 16)][0]
    HA = 64
    ga = pltpu.async_copy(
        tab_hbm.at[pl.ds(src, 1), pl.ds(0, HA)],
        plane_v.at[pl.ds(0, 1), pl.ds(0, HA)], gsem)
    gb = pltpu.async_copy(
        tab_hbm.at[pl.ds(src, 1), pl.ds(HA, H - HA)],
        plane_v.at[pl.ds(0, 1), pl.ds(HA, H - HA)], g2sem)
    ga.wait()
    copies = [
        pltpu.async_copy(
            plane_v.at[pl.ds(0, 1), pl.ds(0, HA)],
            out_hbm.at[pl.ds(wid * C + c, 1), pl.ds(0, HA)], wsem)
        for c in range(C)
    ]
    gb.wait()
    copies += [
        pltpu.async_copy(
            plane_v.at[pl.ds(0, 1), pl.ds(HA, H - HA)],
            out_hbm.at[pl.ds(wid * C + c, 1), pl.ds(HA, H - HA)], wsem)
        for c in range(C)
    ]
    for cp in copies:
        cp.wait()


def kernel(logits, indices):
    tab = logits.reshape(B * C, H, W)
    idx = indices.astype(jnp.int32)
    out = _sc_gather_bcast(tab, idx)
    return out.reshape(B, C, H, W)


# speedup vs baseline: 1.0172x; 1.0042x over previous
import functools

import jax
import jax.numpy as jnp
from jax import lax
from jax.experimental import pallas as pl
from jax.experimental.pallas import tpu as pltpu
from jax.experimental.pallas import tpu_sc as plsc

B, C, H, W = 32, 32, 224, 224

_mesh = plsc.VectorSubcoreMesh(core_axis_name="c", subcore_axis_name="s")


@functools.partial(
    pl.kernel,
    out_type=jax.ShapeDtypeStruct((B * C, H, W), jnp.float32),
    mesh=_mesh,
    scratch_types=[
        pltpu.VMEM((B + 16,), jnp.int32),
        pltpu.VMEM((1, H, W), jnp.float32),
        pltpu.SemaphoreType.DMA,
        pltpu.SemaphoreType.DMA,
    ],
)
def _sc_gather_bcast(tab_hbm, idx_hbm, out_hbm, idx_v, plane_v, gsem, wsem):
    wid = lax.axis_index("s") * 2 + lax.axis_index("c")
    pltpu.sync_copy(idx_hbm, idx_v.at[pl.ds(0, B)])
    src = wid * C + idx_v[pl.ds(wid, 16)][0]
    pltpu.async_copy(tab_hbm.at[pl.ds(src, 1)], plane_v, gsem).wait()
    copies = [
        pltpu.async_copy(plane_v, out_hbm.at[pl.ds(wid * C + c, 1)], wsem)
        for c in range(C)
    ]
    for cp in copies:
        cp.wait()


def kernel(logits, indices):
    tab = logits.reshape(B * C, H, W)
    idx = indices.astype(jnp.int32)
    out = _sc_gather_bcast(tab, idx)
    return out.reshape(B, C, H, W)
